# Initial kernel scaffold; baseline (speedup 1.0000x reference)
import jax
import jax.numpy as jnp
from jax.experimental import pallas as pl


def kernel(contextual, t_W1, t_b1, t_W2, t_b2, c_W1, c_b1, c_W2, c_b2, gumbel_u):
    raise NotImplementedError("write your pallas kernel here")


# speedup vs baseline: 3.2675x; 3.2675x over previous
import functools

import jax
import jax.numpy as jnp
from jax.experimental import pallas as pl

B, C, T, H, E, TAU, K = 32768, 13, 24, 64, 64, 1.0, 8

BLOCK_B = 2048


def _router_body(x_ref, t_W1_ref, t_b1_ref, t_W2t_ref, t_b2_ref,
                 c_W1_ref, c_b1_ref, c_W2_ref, c_b2_ref, u_ref,
                 mask_ref, probs_ref):
    t_W1 = t_W1_ref[...]
    t_b1 = t_b1_ref[...]
    t_w2 = t_W2t_ref[...]
    t_b2 = t_b2_ref[...]
    cols = []
    for c in range(C):
        xc = x_ref[:, c * T:(c + 1) * T]
        h = jax.nn.gelu(jnp.dot(xc, t_W1,
                                preferred_element_type=jnp.float32) + t_b1)
        cols.append(jnp.sum(h * t_w2, axis=1, keepdims=True) + t_b2)
    t_out = jnp.concatenate(cols, axis=1)

    h2 = jax.nn.gelu(jnp.dot(t_out, c_W1_ref[...],
                             preferred_element_type=jnp.float32)
                     + c_b1_ref[...])
    logits = jnp.dot(h2, c_W2_ref[...],
                     preferred_element_type=jnp.float32) + c_b2_ref[...]

    g = -jnp.log(-jnp.log(u_ref[...]))
    s = (logits + g) / TAU
    m = jnp.max(s, axis=1, keepdims=True)
    e = jnp.exp(s - m)
    p = e / jnp.sum(e, axis=1, keepdims=True)
    probs_ref[...] = p

    lane = jax.lax.broadcasted_iota(jnp.int32, p.shape, 1)
    work = p
    mask = jnp.zeros_like(p)
    for _ in range(K):
        mx = jnp.max(work, axis=1, keepdims=True)
        cand = jnp.where(work == mx, lane, E)
        idx = jnp.min(cand, axis=1, keepdims=True)
        sel = lane == idx
        mask = jnp.where(sel, 1.0, mask)
        work = jnp.where(sel, -jnp.inf, work)
    mask_ref[...] = mask


@jax.jit
def kernel(contextual, t_W1, t_b1, t_W2, t_b2, c_W1, c_b1, c_W2, c_b2,
           gumbel_u):
    x2 = contextual.reshape(B, C * T)
    t_W2t = t_W2.reshape(1, H)
    t_b1r = t_b1.reshape(1, H)
    t_b2r = t_b2.reshape(1, 1)
    c_b1r = c_b1.reshape(1, H)
    c_b2r = c_b2.reshape(1, E)

    grid = (B // BLOCK_B,)
    row_blk = lambda i: (i, 0)
    rep_blk = lambda i: (0, 0)
    mask, probs = pl.pallas_call(
        _router_body,
        grid=grid,
        in_specs=[
            pl.BlockSpec((BLOCK_B, C * T), row_blk),
            pl.BlockSpec((T, H), rep_blk),
            pl.BlockSpec((1, H), rep_blk),
            pl.BlockSpec((1, H), rep_blk),
            pl.BlockSpec((1, 1), rep_blk),
            pl.BlockSpec((C, H), rep_blk),
            pl.BlockSpec((1, H), rep_blk),
            pl.BlockSpec((H, E), rep_blk),
            pl.BlockSpec((1, E), rep_blk),
            pl.BlockSpec((BLOCK_B, E), row_blk),
        ],
        out_specs=[
            pl.BlockSpec((BLOCK_B, E), row_blk),
            pl.BlockSpec((BLOCK_B, E), row_blk),
        ],
        out_shape=[
            jax.ShapeDtypeStruct((B, E), jnp.float32),
            jax.ShapeDtypeStruct((B, E), jnp.float32),
        ],
    )(x2, t_W1, t_b1r, t_W2t, t_b2r, c_W1, c_b1r, c_W2, c_b2r, gumbel_u)
    return (mask, probs)


# speedup vs baseline: 4.8536x; 1.4854x over previous
import functools

import jax
import jax.numpy as jnp
from jax.experimental import pallas as pl

B, C, T, H, E, TAU, K = 32768, 13, 24, 64, 64, 1.0, 8

BLOCK_B = 2048


def _router_body(x_ref, W1bd_ref, b1t_ref, W2bd_ref, t_b2_ref,
                 c_W1_ref, c_b1_ref, c_W2_ref, c_b2_ref, u_ref,
                 mask_ref, probs_ref):
    hh = jax.nn.gelu(jnp.dot(x_ref[...], W1bd_ref[...],
                             preferred_element_type=jnp.float32)
                     + b1t_ref[...])
    t_out = jnp.dot(hh, W2bd_ref[...],
                    preferred_element_type=jnp.float32) + t_b2_ref[...]

    h2 = jax.nn.gelu(jnp.dot(t_out, c_W1_ref[...],
                             preferred_element_type=jnp.float32)
                     + c_b1_ref[...])
    logits = jnp.dot(h2, c_W2_ref[...],
                     preferred_element_type=jnp.float32) + c_b2_ref[...]

    g = -jnp.log(-jnp.log(u_ref[...]))
    s = (logits + g) / TAU
    m = jnp.max(s, axis=1, keepdims=True)
    e = jnp.exp(s - m)
    p = e / jnp.sum(e, axis=1, keepdims=True)
    probs_ref[...] = p

    lane = jax.lax.broadcasted_iota(jnp.int32, p.shape, 1)
    work = p
    mask = jnp.zeros_like(p)
    for _ in range(K):
        mx = jnp.max(work, axis=1, keepdims=True)
        cand = jnp.where(work == mx, lane, E)
        idx = jnp.min(cand, axis=1, keepdims=True)
        sel = lane == idx
        mask = jnp.where(sel, 1.0, mask)
        work = jnp.where(sel, -jnp.inf, work)
    mask_ref[...] = mask


@jax.jit
def kernel(contextual, t_W1, t_b1, t_W2, t_b2, c_W1, c_b1, c_W2, c_b2,
           gumbel_u):
    x2 = contextual.reshape(B, C * T)
    W1bd = jnp.kron(jnp.eye(C, dtype=jnp.float32), t_W1)
    W2bd = jnp.kron(jnp.eye(C, dtype=jnp.float32), t_W2)
    b1t = jnp.tile(t_b1, C).reshape(1, C * H)
    t_b2r = jnp.broadcast_to(t_b2, (C,)).reshape(1, C)
    c_b1r = c_b1.reshape(1, H)
    c_b2r = c_b2.reshape(1, E)

    grid = (B // BLOCK_B,)
    row_blk = lambda i: (i, 0)
    rep_blk = lambda i: (0, 0)
    mask, probs = pl.pallas_call(
        _router_body,
        grid=grid,
        in_specs=[
            pl.BlockSpec((BLOCK_B, C * T), row_blk),
            pl.BlockSpec((C * T, C * H), rep_blk),
            pl.BlockSpec((1, C * H), rep_blk),
            pl.BlockSpec((C * H, C), rep_blk),
            pl.BlockSpec((1, C), rep_blk),
            pl.BlockSpec((C, H), rep_blk),
            pl.BlockSpec((1, H), rep_blk),
            pl.BlockSpec((H, E), rep_blk),
            pl.BlockSpec((1, E), rep_blk),
            pl.BlockSpec((BLOCK_B, E), row_blk),
        ],
        out_specs=[
            pl.BlockSpec((BLOCK_B, E), row_blk),
            pl.BlockSpec((BLOCK_B, E), row_blk),
        ],
        out_shape=[
            jax.ShapeDtypeStruct((B, E), jnp.float32),
            jax.ShapeDtypeStruct((B, E), jnp.float32),
        ],
    )(x2, W1bd, b1t, W2bd, t_b2r, c_W1, c_b1r, c_W2, c_b2r, gumbel_u)
    return (mask, probs)


# speedup vs baseline: 5.6172x; 1.1573x over previous
import functools

import jax
import jax.numpy as jnp
from jax import lax
from jax.experimental import pallas as pl
from jax.experimental.pallas import tpu as pltpu
from jax.experimental.pallas import tpu_sc as plsc

B, C, T, H, E, TAU, K = 32768, 13, 24, 64, 64, 1.0, 8

BLOCK_B = 2048


def _router_body(x_ref, W1bd_ref, b1t_ref, W2bd_ref, t_b2_ref,
                 c_W1_ref, c_b1_ref, c_W2_ref, c_b2_ref, u_ref,
                 probs_ref):
    hh = jax.nn.gelu(jnp.dot(x_ref[...], W1bd_ref[...],
                             preferred_element_type=jnp.float32)
                     + b1t_ref[...])
    t_out = jnp.dot(hh, W2bd_ref[...],
                    preferred_element_type=jnp.float32) + t_b2_ref[...]

    h2 = jax.nn.gelu(jnp.dot(t_out, c_W1_ref[...],
                             preferred_element_type=jnp.float32)
                     + c_b1_ref[...])
    logits = jnp.dot(h2, c_W2_ref[...],
                     preferred_element_type=jnp.float32) + c_b2_ref[...]

    g = -jnp.log(-jnp.log(u_ref[...]))
    s = (logits + g) / TAU
    m = jnp.max(s, axis=1, keepdims=True)
    e = jnp.exp(s - m)
    probs_ref[...] = e / jnp.sum(e, axis=1, keepdims=True)


def _tc_probs(x2, W1bd, b1t, W2bd, t_b2r, c_W1, c_b1r, c_W2, c_b2r,
              gumbel_u):
    grid = (B // BLOCK_B,)
    row_blk = lambda i: (i, 0)
    rep_blk = lambda i: (0, 0)
    return pl.pallas_call(
        _router_body,
        grid=grid,
        in_specs=[
            pl.BlockSpec((BLOCK_B, C * T), row_blk),
            pl.BlockSpec((C * T, C * H), rep_blk),
            pl.BlockSpec((1, C * H), rep_blk),
            pl.BlockSpec((C * H, C), rep_blk),
            pl.BlockSpec((1, C), rep_blk),
            pl.BlockSpec((C, H), rep_blk),
            pl.BlockSpec((1, H), rep_blk),
            pl.BlockSpec((H, E), rep_blk),
            pl.BlockSpec((1, E), rep_blk),
            pl.BlockSpec((BLOCK_B, E), row_blk),
        ],
        out_specs=pl.BlockSpec((BLOCK_B, E), row_blk),
        out_shape=jax.ShapeDtypeStruct((B, E), jnp.float32),
    )(x2, W1bd, b1t, W2bd, t_b2r, c_W1, c_b1r, c_W2, c_b2r, gumbel_u)


def _sc_mask(probs):
    info = plsc.get_sparse_core_info()
    NC, NS, L = info.num_cores, info.num_subcores, info.num_lanes
    NW = NC * NS
    RPW = B // NW
    NG = RPW // L
    mesh = plsc.VectorSubcoreMesh(core_axis_name="c", subcore_axis_name="s")

    NV = E // L

    @functools.partial(
        pl.kernel, mesh=mesh,
        out_type=jax.ShapeDtypeStruct((B * E,), jnp.float32),
        scratch_types=[pltpu.VMEM((RPW * E,), jnp.float32)],
        compiler_params=pltpu.CompilerParams(needs_layout_passes=False),
    )
    def mask_kernel(probs_hbm, mask_hbm, slab):
        wid = lax.axis_index("s") * NC + lax.axis_index("c")
        base = wid * (RPW * E)
        pltpu.sync_copy(probs_hbm.at[pl.ds(base, RPW * E)], slab)

        lane = lax.iota(jnp.int32, L)
        top8 = lane < K

        def one_row(r0):
            v = [slab[pl.ds(r0 + i * L, L)] for i in range(NV)]
            s = [plsc.sort_key_val(x, x, descending=True)[0] for x in v]
            m01 = jnp.maximum(s[0], lax.rev(s[1], (0,)))
            m23 = jnp.maximum(s[2], lax.rev(s[3], (0,)))
            m01 = plsc.sort_key_val(m01, m01, descending=True)[0]
            m23 = plsc.sort_key_val(m23, m23, descending=True)[0]
            f = jnp.maximum(m01, lax.rev(m23, (0,)))
            f = plsc.sort_key_val(f, f, descending=True)[0]
            thr = jnp.min(jnp.where(top8, f, jnp.inf))
            gt = [x > thr for x in v]
            eq = [x == thr for x in v]
            eqi = [jnp.where(x, 1, 0) for x in eq]
            ngt = plsc.all_reduce_population_count(gt[0])
            for i in range(1, NV):
                ngt = ngt + plsc.all_reduce_population_count(gt[i])
            need = K - ngt
            carry = jnp.zeros((L,), jnp.int32)
            for i in range(NV):
                excl = plsc.cumsum(eqi[i]) - eqi[i]
                take = jnp.logical_and(eq[i], (carry + excl) < need)
                out = jnp.where(jnp.logical_or(gt[i], take), 1.0, 0.0)
                slab[pl.ds(r0 + i * L, L)] = out
                carry = carry + plsc.all_reduce_population_count(eq[i])

        def row_block(g, carry_):
            r0 = g * (2 * E)
            one_row(r0)
            one_row(r0 + E)
            return carry_

        lax.fori_loop(0, RPW // 2, row_block, 0)
        pltpu.sync_copy(slab, mask_hbm.at[pl.ds(base, RPW * E)])

    return mask_kernel(probs.reshape(B * E)).reshape(B, E)


@jax.jit
def kernel(contextual, t_W1, t_b1, t_W2, t_b2, c_W1, c_b1, c_W2, c_b2,
           gumbel_u):
    x2 = contextual.reshape(B, C * T)
    W1bd = jnp.kron(jnp.eye(C, dtype=jnp.float32), t_W1)
    W2bd = jnp.kron(jnp.eye(C, dtype=jnp.float32), t_W2)
    b1t = jnp.tile(t_b1, C).reshape(1, C * H)
    t_b2r = jnp.broadcast_to(t_b2, (C,)).reshape(1, C)
    c_b1r = c_b1.reshape(1, H)
    c_b2r = c_b2.reshape(1, E)

    probs = _tc_probs(x2, W1bd, b1t, W2bd, t_b2r, c_W1, c_b1r, c_W2,
                      c_b2r, gumbel_u)
    mask = _sc_mask(probs)
    return (mask, probs)


# speedup vs baseline: 6.1750x; 1.0993x over previous
import functools

import jax
import jax.numpy as jnp
from jax import lax
from jax.experimental import pallas as pl
from jax.experimental.pallas import tpu as pltpu
from jax.experimental.pallas import tpu_sc as plsc

B, C, T, H, E, TAU, K = 32768, 13, 24, 64, 64, 1.0, 8

BLOCK_B = 2048


def _router_body(x_ref, W1bd_ref, b1t_ref, W2bd_ref, t_b2_ref,
                 c_W1_ref, c_b1_ref, c_W2_ref, c_b2_ref, u_ref,
                 probs_ref):
    hh = jax.nn.gelu(jnp.dot(x_ref[...], W1bd_ref[...],
                             preferred_element_type=jnp.float32)
                     + b1t_ref[...])
    t_out = jnp.dot(hh, W2bd_ref[...],
                    preferred_element_type=jnp.float32) + t_b2_ref[...]

    h2 = jax.nn.gelu(jnp.dot(t_out, c_W1_ref[...],
                             preferred_element_type=jnp.float32)
                     + c_b1_ref[...])
    logits = jnp.dot(h2, c_W2_ref[...],
                     preferred_element_type=jnp.float32) + c_b2_ref[...]

    g = -jnp.log(-jnp.log(u_ref[...]))
    s = (logits + g) / TAU
    m = jnp.max(s, axis=1, keepdims=True)
    e = jnp.exp(s - m)
    probs_ref[...] = e / jnp.sum(e, axis=1, keepdims=True)


def _tc_probs(x2, W1bd, b1t, W2bd, t_b2r, c_W1, c_b1r, c_W2, c_b2r,
              gumbel_u):
    grid = (B // BLOCK_B,)
    row_blk = lambda i: (i, 0)
    rep_blk = lambda i: (0, 0)
    return pl.pallas_call(
        _router_body,
        grid=grid,
        in_specs=[
            pl.BlockSpec((BLOCK_B, C * T), row_blk),
            pl.BlockSpec((C * T, C * H), rep_blk),
            pl.BlockSpec((1, C * H), rep_blk),
            pl.BlockSpec((C * H, C), rep_blk),
            pl.BlockSpec((1, C), rep_blk),
            pl.BlockSpec((C, H), rep_blk),
            pl.BlockSpec((1, H), rep_blk),
            pl.BlockSpec((H, E), rep_blk),
            pl.BlockSpec((1, E), rep_blk),
            pl.BlockSpec((BLOCK_B, E), row_blk),
        ],
        out_specs=pl.BlockSpec((BLOCK_B, E), row_blk),
        out_shape=jax.ShapeDtypeStruct((B, E), jnp.float32),
    )(x2, W1bd, b1t, W2bd, t_b2r, c_W1, c_b1r, c_W2, c_b2r, gumbel_u)


def _sc_mask(probs):
    info = plsc.get_sparse_core_info()
    NC, NS, L = info.num_cores, info.num_subcores, info.num_lanes
    NW = NC * NS
    RPW = B // NW
    NG = RPW // L
    mesh = plsc.VectorSubcoreMesh(core_axis_name="c", subcore_axis_name="s")

    NV = E // L

    @functools.partial(
        pl.kernel, mesh=mesh,
        out_type=jax.ShapeDtypeStruct((B, E), jnp.float32),
        scratch_types=[pltpu.VMEM((RPW, E), jnp.float32)],
        compiler_params=pltpu.CompilerParams(needs_layout_passes=False),
    )
    def mask_kernel(probs_hbm, mask_hbm, slab):
        wid = lax.axis_index("s") * NC + lax.axis_index("c")
        base = wid * RPW
        pltpu.sync_copy(probs_hbm.at[pl.ds(base, RPW)], slab)

        lane = lax.iota(jnp.int32, L)
        top8 = lane < K

        def one_row(rr):
            v = [slab[rr, pl.ds(i * L, L)] for i in range(NV)]
            s = [plsc.sort_key_val(x, x, descending=True)[0] for x in v]
            m01 = jnp.maximum(s[0], lax.rev(s[1], (0,)))
            m23 = jnp.maximum(s[2], lax.rev(s[3], (0,)))
            m01 = plsc.sort_key_val(m01, m01, descending=True)[0]
            m23 = plsc.sort_key_val(m23, m23, descending=True)[0]
            f = jnp.maximum(m01, lax.rev(m23, (0,)))
            f = plsc.sort_key_val(f, f, descending=True)[0]
            thr = jnp.min(jnp.where(top8, f, jnp.inf))
            gt = [x > thr for x in v]
            eq = [x == thr for x in v]
            eqi = [jnp.where(x, 1, 0) for x in eq]
            ngt = plsc.all_reduce_population_count(gt[0])
            for i in range(1, NV):
                ngt = ngt + plsc.all_reduce_population_count(gt[i])
            need = K - ngt
            carry = jnp.zeros((L,), jnp.int32)
            for i in range(NV):
                excl = plsc.cumsum(eqi[i]) - eqi[i]
                take = jnp.logical_and(eq[i], (carry + excl) < need)
                out = jnp.where(jnp.logical_or(gt[i], take), 1.0, 0.0)
                slab[rr, pl.ds(i * L, L)] = out
                carry = carry + plsc.all_reduce_population_count(eq[i])

        def row_block(g, carry_):
            one_row(2 * g)
            one_row(2 * g + 1)
            return carry_

        lax.fori_loop(0, RPW // 2, row_block, 0)
        pltpu.sync_copy(slab, mask_hbm.at[pl.ds(base, RPW)])

    return mask_kernel(probs)


@jax.jit
def kernel(contextual, t_W1, t_b1, t_W2, t_b2, c_W1, c_b1, c_W2, c_b2,
           gumbel_u):
    x2 = contextual.reshape(B, C * T)
    W1bd = jnp.kron(jnp.eye(C, dtype=jnp.float32), t_W1)
    W2bd = jnp.kron(jnp.eye(C, dtype=jnp.float32), t_W2)
    b1t = jnp.tile(t_b1, C).reshape(1, C * H)
    t_b2r = jnp.broadcast_to(t_b2, (C,)).reshape(1, C)
    c_b1r = c_b1.reshape(1, H)
    c_b2r = c_b2.reshape(1, E)

    probs = _tc_probs(x2, W1bd, b1t, W2bd, t_b2r, c_W1, c_b1r, c_W2,
                      c_b2r, gumbel_u)
    mask = _sc_mask(probs)
    return (mask, probs)


# speedup vs baseline: 10.7028x; 1.7332x over previous
import functools

import jax
import jax.numpy as jnp
from jax import lax
from jax.experimental import pallas as pl
from jax.experimental.pallas import tpu as pltpu
from jax.experimental.pallas import tpu_sc as plsc

B, C, T, H, E, TAU, K = 32768, 13, 24, 64, 64, 1.0, 8

BLOCK_B = 2048


def _router_body(xT_ref, W1bdT_ref, b1tT_ref, W2bdT_ref, t_b2T_ref,
                 c_W1T_ref, cb1T_ref, c_W2T_ref, c_b2T_ref, uT_ref,
                 probsT_ref):
    hhT = jax.nn.gelu(jnp.dot(W1bdT_ref[...], xT_ref[...],
                              preferred_element_type=jnp.float32)
                      + b1tT_ref[...])
    t_outT = jnp.dot(W2bdT_ref[...], hhT,
                     preferred_element_type=jnp.float32) + t_b2T_ref[...]
    h2T = jax.nn.gelu(jnp.dot(c_W1T_ref[...], t_outT,
                              preferred_element_type=jnp.float32)
                      + cb1T_ref[...])
    logitsT = jnp.dot(c_W2T_ref[...], h2T,
                      preferred_element_type=jnp.float32) + c_b2T_ref[...]

    gT = -jnp.log(-jnp.log(uT_ref[...]))
    sT = (logitsT + gT) / TAU
    mT = jnp.max(sT, axis=0, keepdims=True)
    eT = jnp.exp(sT - mT)
    probsT_ref[...] = eT / jnp.sum(eT, axis=0, keepdims=True)


def _tc_probs_t(xT, W1bdT, b1tT, W2bdT, t_b2T, c_W1T, cb1T, c_W2T, c_b2T,
                uT):
    grid = (B // BLOCK_B,)
    col_blk = lambda i: (0, i)
    rep_blk = lambda i: (0, 0)
    return pl.pallas_call(
        _router_body,
        grid=grid,
        in_specs=[
            pl.BlockSpec((C * T, BLOCK_B), col_blk),
            pl.BlockSpec((C * H, C * T), rep_blk),
            pl.BlockSpec((C * H, 1), rep_blk),
            pl.BlockSpec((C, C * H), rep_blk),
            pl.BlockSpec((C, 1), rep_blk),
            pl.BlockSpec((E, C), rep_blk),
            pl.BlockSpec((E, 1), rep_blk),
            pl.BlockSpec((E, E), rep_blk),
            pl.BlockSpec((E, 1), rep_blk),
            pl.BlockSpec((E, BLOCK_B), col_blk),
        ],
        out_specs=pl.BlockSpec((E, BLOCK_B), col_blk),
        out_shape=jax.ShapeDtypeStruct((E, B), jnp.float32),
    )(xT, W1bdT, b1tT, W2bdT, t_b2T, c_W1T, cb1T, c_W2T, c_b2T, uT)


def _sc_mask_t(probsT):
    info = plsc.get_sparse_core_info()
    NC, NS, L = info.num_cores, info.num_subcores, info.num_lanes
    NW = NC * NS
    TPW = B // NW
    NG = TPW // L

    mesh = plsc.VectorSubcoreMesh(core_axis_name="c", subcore_axis_name="s")

    @functools.partial(
        pl.kernel, mesh=mesh,
        out_type=jax.ShapeDtypeStruct((E, B), jnp.float32),
        scratch_types=[pltpu.VMEM((E, TPW), jnp.float32)],
        compiler_params=pltpu.CompilerParams(needs_layout_passes=False),
    )
    def mask_kernel(probsT_hbm, maskT_hbm, slab):
        wid = lax.axis_index("s") * NC + lax.axis_index("c")
        base = wid * TPW
        pltpu.sync_copy(probsT_hbm.at[:, pl.ds(base, TPW)], slab)

        def group(g, carry_):
            t0 = g * L
            m = [jnp.full((L,), -jnp.inf, jnp.float32) for _ in range(K)]
            for e in range(E):
                v = slab[e, pl.ds(t0, L)]
                for lvl in range(K):
                    hi = jnp.maximum(m[lvl], v)
                    v = jnp.minimum(m[lvl], v)
                    m[lvl] = hi
            thr = m[K - 1]
            cnt = jnp.zeros((L,), jnp.float32)
            for e in range(E):
                v = slab[e, pl.ds(t0, L)]
                cnt = cnt + jnp.where(v > thr, 1.0, 0.0)
            need = float(K) - cnt
            eqc = jnp.zeros((L,), jnp.float32)
            for e in range(E):
                v = slab[e, pl.ds(t0, L)]
                gt = v > thr
                eq = v == thr
                take = jnp.logical_and(eq, eqc < need)
                slab[e, pl.ds(t0, L)] = jnp.where(
                    jnp.logical_or(gt, take), 1.0, 0.0)
                eqc = eqc + jnp.where(eq, 1.0, 0.0)
            return carry_

        lax.fori_loop(0, NG, group, 0)
        pltpu.sync_copy(slab, maskT_hbm.at[:, pl.ds(base, TPW)])

    return mask_kernel(probsT)


@jax.jit
def kernel(contextual, t_W1, t_b1, t_W2, t_b2, c_W1, c_b1, c_W2, c_b2,
           gumbel_u):
    xT = contextual.transpose(1, 2, 0).reshape(C * T, B)
    uT = gumbel_u.T

    eye = jnp.eye(C, dtype=jnp.float32)
    W1bdT = jnp.kron(eye, t_W1).T
    b1tT = jnp.tile(t_b1, C).reshape(C * H, 1)
    W2bdT = jnp.kron(eye, t_W2).T
    t_b2T = jnp.broadcast_to(t_b2, (C,)).reshape(C, 1)
    c_W1T = c_W1.T
    cb1T = c_b1.reshape(E, 1)
    c_W2T = c_W2.T
    c_b2T = c_b2.reshape(E, 1)

    probsT = _tc_probs_t(xT, W1bdT, b1tT, W2bdT, t_b2T, c_W1T, cb1T,
                         c_W2T, c_b2T, uT)
    maskT = _sc_mask_t(probsT)
    return (maskT.T, probsT.T)
